# Initial kernel scaffold; baseline (speedup 1.0000x reference)
import jax
import jax.numpy as jnp
from jax.experimental import pallas as pl


def kernel(x, y):
    raise NotImplementedError("write your pallas kernel here")


# speedup vs baseline: 15.5937x; 15.5937x over previous
import functools

import jax
import jax.numpy as jnp
from jax import lax
from jax.experimental import pallas as pl
from jax.experimental.pallas import tpu as pltpu
from jax.experimental.pallas import tpu_sc as plsc

_N = 8388608
_NC = 2
_NS = 16
_NW = _NC * _NS
_L = 16
_C = 8192
_PER_W = _N // _NW
_NCHUNK = _PER_W // _C


def _sc_body(x_hbm, y_hbm, out_hbm, part_hbm, xb, yb, ob, pb):
    wid = lax.axis_index("s") * _NC + lax.axis_index("c")
    base = wid * _PER_W

    zero = jnp.zeros((_L,), jnp.float32)
    one = jnp.ones((_L,), jnp.float32)

    def chunk_body(g, carry):
        c0, c2 = carry
        off = base + g * _C
        pltpu.sync_copy(x_hbm.at[pl.ds(off, _C)], xb)
        pltpu.sync_copy(y_hbm.at[pl.ds(off, _C)], yb)

        def inner(i, cc):
            a0, a2 = cc
            xv = xb[pl.ds(i * _L, _L)]
            yv = yb[pl.ds(i * _L, _L)]
            ov = xv + yv
            ob[pl.ds(i * _L, _L)] = ov
            a0 = a0 + jnp.where(ov == 0.0, one, zero)
            a2 = a2 + jnp.where(ov == 2.0, one, zero)
            return (a0, a2)

        c0, c2 = lax.fori_loop(0, _C // _L, inner, (c0, c2))
        pltpu.sync_copy(ob, out_hbm.at[pl.ds(off, _C)])
        return (c0, c2)

    c0, c2 = lax.fori_loop(0, _NCHUNK, chunk_body, (zero, zero))
    pb[pl.ds(0, _L)] = c0
    pb[pl.ds(_L, _L)] = c2
    pltpu.sync_copy(pb, part_hbm.at[wid])


_sc_call = functools.partial(
    pl.kernel,
    out_type=[
        jax.ShapeDtypeStruct((_N,), jnp.float32),
        jax.ShapeDtypeStruct((_NW, 2 * _L), jnp.float32),
    ],
    mesh=plsc.VectorSubcoreMesh(core_axis_name="c", subcore_axis_name="s"),
    scratch_types=[
        pltpu.VMEM((_C,), jnp.float32),
        pltpu.VMEM((_C,), jnp.float32),
        pltpu.VMEM((_C,), jnp.float32),
        pltpu.VMEM((2 * _L,), jnp.float32),
    ],
)(_sc_body)


@jax.jit
def kernel(x, y):
    out, parts = _sc_call(x, y)
    c0 = jnp.sum(parts[:, :_L])
    c2 = jnp.sum(parts[:, _L:])
    n = jnp.float32(_N)
    denom0 = n - c2
    denom1 = n - c0
    valid0 = denom0 > 0
    valid1 = denom1 > 0
    iou0 = jnp.where(valid0, c0 / jnp.where(valid0, denom0, 1.0), 0.0)
    iou1 = jnp.where(valid1, c2 / jnp.where(valid1, denom1, 1.0), 0.0)
    n_valid = jnp.maximum(
        valid0.astype(jnp.float32) + valid1.astype(jnp.float32), 1.0
    )
    miou = (iou0 + iou1) / n_valid
    return out, miou


# speedup vs baseline: 28.7381x; 1.8429x over previous
import functools

import jax
import jax.numpy as jnp
from jax import lax
from jax.experimental import pallas as pl
from jax.experimental.pallas import tpu as pltpu
from jax.experimental.pallas import tpu_sc as plsc

_N = 8388608
_NC = 2
_NS = 16
_NW = _NC * _NS
_L = 16
_C = 16384
_PER_W = _N // _NW
_NCHUNK = _PER_W // _C


def _sc_body(x_hbm, y_hbm, out_hbm, part_hbm,
             xb0, yb0, ob0, xb1, yb1, ob1, pb,
             sx0, sy0, so0, sx1, sy1, so1):
    wid = lax.axis_index("s") * _NC + lax.axis_index("c")
    base = wid * _PER_W

    bufs = ((xb0, yb0, ob0, sx0, sy0, so0),
            (xb1, yb1, ob1, sx1, sy1, so1))

    zero = jnp.zeros((_L,), jnp.float32)
    one = jnp.ones((_L,), jnp.float32)

    def start_load(g, b):
        xb, yb, _, sx, sy, _ = bufs[b]
        off = base + g * _C
        pltpu.make_async_copy(x_hbm.at[pl.ds(off, _C)], xb, sx).start()
        pltpu.make_async_copy(y_hbm.at[pl.ds(off, _C)], yb, sy).start()

    def wait_load(b):
        xb, yb, _, sx, sy, _ = bufs[b]
        pltpu.make_async_copy(x_hbm.at[pl.ds(0, _C)], xb, sx).wait()
        pltpu.make_async_copy(y_hbm.at[pl.ds(0, _C)], yb, sy).wait()

    def start_store(g, b):
        _, _, ob, _, _, so = bufs[b]
        off = base + g * _C
        pltpu.make_async_copy(ob, out_hbm.at[pl.ds(off, _C)], so).start()

    def wait_store(b):
        _, _, ob, _, _, so = bufs[b]
        pltpu.make_async_copy(ob, out_hbm.at[pl.ds(0, _C)], so).wait()

    def compute(b, carry):
        xb, yb, ob, _, _, _ = bufs[b]

        @plsc.parallel_loop(0, _C // _L, carry=carry, unroll=8)
        def body(i, cc):
            a0, a2 = cc
            xv = xb[pl.ds(i * _L, _L)]
            yv = yb[pl.ds(i * _L, _L)]
            ov = xv + yv
            ob[pl.ds(i * _L, _L)] = ov
            a0 = a0 + jnp.where(ov == 0.0, one, zero)
            a2 = a2 + jnp.where(ov == 2.0, one, zero)
            return (a0, a2)

        return body

    start_load(0, 0)
    start_load(1, 1)

    def jbody(j, carry):
        for b in range(2):
            g = 2 * j + b
            wait_load(b)

            @pl.when(j >= 1)
            def _():
                wait_store(b)

            carry = compute(b, carry)
            start_store(g, b)

            @pl.when(j < _NCHUNK // 2 - 1)
            def _():
                start_load(g + 2, b)
        return carry

    c0, c2 = lax.fori_loop(0, _NCHUNK // 2, jbody, (zero, zero))
    wait_store(0)
    wait_store(1)

    pb[pl.ds(0, _L)] = c0
    pb[pl.ds(_L, _L)] = c2
    pltpu.sync_copy(pb, part_hbm.at[wid])


_sc_call = functools.partial(
    pl.kernel,
    out_type=[
        jax.ShapeDtypeStruct((_N,), jnp.float32),
        jax.ShapeDtypeStruct((_NW, 2 * _L), jnp.float32),
    ],
    mesh=plsc.VectorSubcoreMesh(core_axis_name="c", subcore_axis_name="s"),
    scratch_types=[
        pltpu.VMEM((_C,), jnp.float32),
        pltpu.VMEM((_C,), jnp.float32),
        pltpu.VMEM((_C,), jnp.float32),
        pltpu.VMEM((_C,), jnp.float32),
        pltpu.VMEM((_C,), jnp.float32),
        pltpu.VMEM((_C,), jnp.float32),
        pltpu.VMEM((2 * _L,), jnp.float32),
        pltpu.SemaphoreType.DMA,
        pltpu.SemaphoreType.DMA,
        pltpu.SemaphoreType.DMA,
        pltpu.SemaphoreType.DMA,
        pltpu.SemaphoreType.DMA,
        pltpu.SemaphoreType.DMA,
    ],
)(_sc_body)


@jax.jit
def kernel(x, y):
    out, parts = _sc_call(x, y)
    c0 = jnp.sum(parts[:, :_L])
    c2 = jnp.sum(parts[:, _L:])
    n = jnp.float32(_N)
    denom0 = n - c2
    denom1 = n - c0
    valid0 = denom0 > 0
    valid1 = denom1 > 0
    iou0 = jnp.where(valid0, c0 / jnp.where(valid0, denom0, 1.0), 0.0)
    iou1 = jnp.where(valid1, c2 / jnp.where(valid1, denom1, 1.0), 0.0)
    n_valid = jnp.maximum(
        valid0.astype(jnp.float32) + valid1.astype(jnp.float32), 1.0
    )
    miou = (iou0 + iou1) / n_valid
    return out, miou
